# Initial kernel scaffold; baseline (speedup 1.0000x reference)
import jax
import jax.numpy as jnp
from jax.experimental import pallas as pl


def kernel(edge_index, etype, norm, in_edges_mask, bp_edge_index, bp_etype, n_embds, rel_embds, degree_basis, degree_weights_agg, sub_W, sub_b, whp_W, whp_b, l0_WO, l0_WI, l0_WS, l0_WR, l0_loop, g0_W, g0_b, l1_WO, l1_WI, l1_WS, l1_WR, l1_loop, g1_W, g1_b):
    raise NotImplementedError("write your pallas kernel here")


# speedup vs baseline: 2.9345x; 2.9345x over previous
import math

import jax
import jax.numpy as jnp
from jax import lax
from jax.experimental import pallas as pl
from jax.experimental.pallas import tpu as pltpu
from jax.experimental.pallas import tpu_sc as plsc

_NUM_ENT = 10000
_NUM_REL = 50
_ENT_DIM = 256
_INIT_DIM = 128
_BASIS_DIM = 64
_N_EDGES = 160000
_BN_EPS = 1e-5
_SCALE = (1.0 / 3.0) / math.sqrt(1.0 + _BN_EPS)

_NC, _NS = 2, 16
_NW = _NC * _NS
_ROW = 128
_EROWS = _N_EDGES // _ROW
_RBLK = 1000
_CNT_COL = _BASIS_DIM
_DEG_COL = _BASIS_DIM + 1

_f32 = jnp.float32
_i32 = jnp.int32


def _sc1_body(bpd, bpe, ttbl, dst, z128, degones_in,
              bp_out,
              acc, degones, bpd_b, bpe_b, dst_b, vals, gsem):
    cid = lax.axis_index("c")
    sid = lax.axis_index("s")
    wid = sid * _NC + cid

    zlo = 8 * ((1250 * sid) // 16)
    pltpu.sync_copy(z128, acc.at[pl.ds(zlo, 632)])
    pltpu.sync_copy(degones_in, degones)

    lo = (_EROWS * wid) // _NW
    hi = (_EROWS * (wid + 1)) // _NW
    nr = hi - lo

    plsc.subcore_barrier()

    def _bp_row(m, c):
        k = lo + m
        cpd = pltpu.async_copy(bpd.at[pl.ds(k, 1)], bpd_b, gsem)
        cpe = pltpu.async_copy(bpe.at[pl.ds(k, 1)], bpe_b, gsem)
        cpt = pltpu.async_copy(dst.at[pl.ds(k, 1)], dst_b, gsem)
        cpd.wait()
        cpe.wait()
        cpt.wait()
        pltpu.async_copy(ttbl.at[bpe_b.at[0, 0]], vals, gsem).wait()
        pltpu.sync_copy(vals, acc.at[bpd_b.at[0, 0]], add=True)
        pltpu.sync_copy(degones, acc.at[dst_b.at[0, 0]], add=True)
        return c
    lax.fori_loop(0, nr, _bp_row, None)

    plsc.subcore_barrier()

    @pl.when(cid == 0)
    def _():
        pltpu.sync_copy(acc.at[pl.ds(zlo, 632)],
                        bp_out.at[0, pl.ds(zlo, 632)])

    @pl.when(cid == 1)
    def _():
        pltpu.sync_copy(acc.at[pl.ds(zlo, 632)],
                        bp_out.at[1, pl.ds(zlo, 632)])


def _sc1(bpd, bpe, ttbl, dst, z128, degones_in):
    mesh = plsc.VectorSubcoreMesh(core_axis_name="c", subcore_axis_name="s",
                                  num_cores=_NC, num_subcores=_NS)
    fn = pl.kernel(
        _sc1_body,
        out_type=jax.ShapeDtypeStruct((_NC, _NUM_ENT, 128), _f32),
        mesh=mesh,
        scratch_types=[
            pltpu.VMEM_SHARED((_NUM_ENT, 128), _f32),
            pltpu.VMEM((_ROW, 128), _f32),
            pltpu.VMEM((1, 1, _ROW), _i32),
            pltpu.VMEM((1, 1, _ROW), _i32),
            pltpu.VMEM((1, 1, _ROW), _i32),
            pltpu.VMEM((_ROW, 128), _f32),
            pltpu.SemaphoreType.DMA,
        ],
    )
    return fn(bpd, bpe, ttbl, dst, z128, degones_in)


def _sc_edge_body(src, dst, et, nmA, nmM, nh0, nh1, dns, rh0, rh1, z128,
                  fcio, gout,
                  acc, src_b, dst_b, et_b, nmxbuf, rows, rvals,
                  gsem):
    cid = lax.axis_index("c")
    sid = lax.axis_index("s")
    wid = sid * _NC + cid

    lo = (_EROWS * sid) // _NS
    hi = (_EROWS * (sid + 1)) // _NS
    nr = hi - lo

    zlo = 8 * ((1250 * sid) // 16)

    def _fc_pass(tbl, rtb, nmtbl, slot):
        pltpu.sync_copy(z128, acc.at[pl.ds(zlo, 632)])
        plsc.subcore_barrier()

        def _row(m, c):
            k = lo + m
            cps = pltpu.async_copy(src.at[pl.ds(k, 1)], src_b, gsem)
            cpd = pltpu.async_copy(dst.at[pl.ds(k, 1)], dst_b, gsem)
            cpe = pltpu.async_copy(et.at[pl.ds(k, 1)], et_b, gsem)
            cps.wait()
            cpd.wait()
            cpe.wait()
            cp1 = pltpu.async_copy(tbl.at[src_b.at[0, 0]], rows, gsem)
            cp2 = pltpu.async_copy(rtb.at[et_b.at[0, 0]], rvals, gsem)
            cp3 = pltpu.async_copy(nmtbl.at[pl.ds(k, 1)], nmxbuf, gsem)
            cp1.wait()
            cp2.wait()
            cp3.wait()

            def _edge(e, c2):
                nv = nmxbuf[0, e, pl.ds(0, 16)]
                for j in range(8):
                    s = pl.ds(16 * j, 16)
                    rows[e, s] = rows[e, s] * rvals[e, s] * nv
                return c2
            lax.fori_loop(0, _ROW, _edge, None)
            pltpu.sync_copy(rows, acc.at[dst_b.at[0, 0]], add=True)
            return c
        lax.fori_loop(0, nr, _row, None)
        plsc.subcore_barrier()
        pltpu.sync_copy(acc.at[pl.ds(zlo, 632)],
                        fcio.at[cid, slot, pl.ds(zlo, 632)])
        plsc.subcore_barrier()

    def _g_pass():
        pltpu.sync_copy(z128, acc.at[pl.ds(zlo, 632)])
        plsc.subcore_barrier()

        lo_g = (_EROWS * wid) // _NW
        hi_g = (_EROWS * (wid + 1)) // _NW

        def _row(m, c):
            k = lo_g + m
            cps = pltpu.async_copy(src.at[pl.ds(k, 1)], src_b, gsem)
            cpd = pltpu.async_copy(dst.at[pl.ds(k, 1)], dst_b, gsem)
            cps.wait()
            cpd.wait()
            pltpu.async_copy(dns.at[src_b.at[0, 0]], rows, gsem).wait()
            pltpu.sync_copy(rows, acc.at[dst_b.at[0, 0]], add=True)
            return c
        lax.fori_loop(0, hi_g - lo_g, _row, None)
        plsc.subcore_barrier()
        pltpu.sync_copy(acc.at[pl.ds(zlo, 632)],
                        gout.at[cid, pl.ds(zlo, 632)])

    @pl.when(cid == 0)
    def _():
        _fc_pass(nh0, rh0, nmA, 0)
        _fc_pass(nh0, rh0, nmM, 1)

    @pl.when(cid == 1)
    def _():
        _fc_pass(nh1, rh1, nmA, 0)
        _fc_pass(nh1, rh1, nmM, 1)

    _g_pass()


def _sc_edge(src, dst, et, nmA, nmM, nh0, nh1, dns, rh0, rh1, z128):
    mesh = plsc.VectorSubcoreMesh(core_axis_name="c", subcore_axis_name="s",
                                  num_cores=_NC, num_subcores=_NS)
    fn = pl.kernel(
        _sc_edge_body,
        out_type=(
            jax.ShapeDtypeStruct((_NC, 2, _NUM_ENT, 128), _f32),
            jax.ShapeDtypeStruct((_NC, _NUM_ENT, 128), _f32),
        ),
        mesh=mesh,
        scratch_types=[
            pltpu.VMEM_SHARED((_NUM_ENT, 128), _f32),
            pltpu.VMEM((1, 1, _ROW), _i32),
            pltpu.VMEM((1, 1, _ROW), _i32),
            pltpu.VMEM((1, 1, _ROW), _i32),
            pltpu.VMEM((1, _ROW, 16), _f32),
            pltpu.VMEM((_ROW, 128), _f32),
            pltpu.VMEM((_ROW, 128), _f32),
            pltpu.SemaphoreType.DMA,
        ],
    )
    return fn(src, dst, et, nmA, nmM, nh0, nh1, dns, rh0, rh1, z128)


def _dot(a, b):
    return lax.dot_general(a, b, (((1,), (0,)), ((), ())),
                           precision=lax.Precision.HIGHEST,
                           preferred_element_type=_f32)


def _tc1_body(bp0, bp1, ne, basis, swa, swb, sb, nin_o, dns_o, dinv_o):
    sums = bp0[...] + bp1[...]
    cnt = jnp.maximum(sums[:, _CNT_COL:_CNT_COL + 1], 1.0)
    coef = sums[:, :_BASIS_DIM] / cnt
    dn0 = _dot(coef, basis[...])
    dinv = lax.rsqrt(sums[:, _DEG_COL:_DEG_COL + 1] + 1.0)
    nin_o[...] = _dot(ne[...], swa[...]) + _dot(dn0, swb[...]) + sb[...]
    dns_o[...] = dinv * dn0
    dinv_o[...] = jnp.broadcast_to(dinv, dinv_o.shape)


def _tc1(bp0, bp1, ne, basis, swa, swb, sb):
    R = _RBLK
    return pl.pallas_call(
        _tc1_body,
        grid=(_NUM_ENT // R,),
        in_specs=[
            pl.BlockSpec((R, 128), lambda i: (i, 0)),
            pl.BlockSpec((R, 128), lambda i: (i, 0)),
            pl.BlockSpec((R, _ENT_DIM), lambda i: (i, 0)),
            pl.BlockSpec((_BASIS_DIM, _INIT_DIM), lambda i: (0, 0)),
            pl.BlockSpec((_ENT_DIM, _ENT_DIM), lambda i: (0, 0)),
            pl.BlockSpec((_INIT_DIM, _ENT_DIM), lambda i: (0, 0)),
            pl.BlockSpec((1, _ENT_DIM), lambda i: (0, 0)),
        ],
        out_specs=[
            pl.BlockSpec((R, _ENT_DIM), lambda i: (i, 0)),
            pl.BlockSpec((R, _INIT_DIM), lambda i: (i, 0)),
            pl.BlockSpec((R, _INIT_DIM), lambda i: (i, 0)),
        ],
        out_shape=[
            jax.ShapeDtypeStruct((_NUM_ENT, _ENT_DIM), _f32),
            jax.ShapeDtypeStruct((_NUM_ENT, _INIT_DIM), _f32),
            jax.ShapeDtypeStruct((_NUM_ENT, _INIT_DIM), _f32),
        ],
    )(bp0, bp1, ne, basis, swa, swb, sb)


def _tc_layer_body(final, accA, accM, accG, nin, dns, dinvb,
                   WI, WO, WS, lrow, gW, gb, wa, wb, wbias, *outs):
    ce = (_dot(accA[...], WO[...]) + _dot(accM[...], WI[...] - WO[...])
          + _dot(nin[...] * lrow[...], WS[...]))
    n_out = jnp.tanh(ce * _SCALE)
    dinv = dinvb[...]
    dn_new = jnp.tanh(dinv * _dot(dns[...] + accG[...], gW[...]) + gb[...])
    outs[0][...] = _dot(n_out, wa[...]) + _dot(dn_new, wb[...]) + wbias[...]
    if not final:
        outs[1][...] = dinv * dn_new


def _tc_layer(final, accA, accM, accG, nin, dns, dinvb,
              WI, WO, WS, lrow, gW, gb, wa, wb, wbias):
    R = _RBLK
    out_specs = [pl.BlockSpec((R, _ENT_DIM), lambda i: (i, 0))]
    out_shape = [jax.ShapeDtypeStruct((_NUM_ENT, _ENT_DIM), _f32)]
    if not final:
        out_specs.append(pl.BlockSpec((R, _INIT_DIM), lambda i: (i, 0)))
        out_shape.append(jax.ShapeDtypeStruct((_NUM_ENT, _INIT_DIM), _f32))
    body = lambda *a: _tc_layer_body(final, *a)
    return pl.pallas_call(
        body,
        grid=(_NUM_ENT // R,),
        in_specs=[
            pl.BlockSpec((R, _ENT_DIM), lambda i: (i, 0)),
            pl.BlockSpec((R, _ENT_DIM), lambda i: (i, 0)),
            pl.BlockSpec((R, _INIT_DIM), lambda i: (i, 0)),
            pl.BlockSpec((R, _ENT_DIM), lambda i: (i, 0)),
            pl.BlockSpec((R, _INIT_DIM), lambda i: (i, 0)),
            pl.BlockSpec((R, _INIT_DIM), lambda i: (i, 0)),
            pl.BlockSpec((_ENT_DIM, _ENT_DIM), lambda i: (0, 0)),
            pl.BlockSpec((_ENT_DIM, _ENT_DIM), lambda i: (0, 0)),
            pl.BlockSpec((_ENT_DIM, _ENT_DIM), lambda i: (0, 0)),
            pl.BlockSpec((1, _ENT_DIM), lambda i: (0, 0)),
            pl.BlockSpec((_INIT_DIM, _INIT_DIM), lambda i: (0, 0)),
            pl.BlockSpec((1, _INIT_DIM), lambda i: (0, 0)),
            pl.BlockSpec((_ENT_DIM, _ENT_DIM), lambda i: (0, 0)),
            pl.BlockSpec((_INIT_DIM, _ENT_DIM), lambda i: (0, 0)),
            pl.BlockSpec((1, _ENT_DIM), lambda i: (0, 0)),
        ],
        out_specs=out_specs,
        out_shape=out_shape,
    )(accA, accM, accG, nin, dns, dinvb,
      WI, WO, WS, lrow, gW, gb, wa, wb, wbias)


def _rk_body(rel, W0, W1, dwagg, r1_o, r2_o, t_o):
    r1 = _dot(rel[...], W0[...])
    r1_o[...] = r1
    r2_o[...] = _dot(r1, W1[...])
    ridx = lax.broadcasted_iota(_i32, (_NUM_REL, 1), 0).astype(_f32)
    col = lax.broadcasted_iota(_i32, (_NUM_REL, 128 - _BASIS_DIM), 1)
    t_o[:, :_BASIS_DIM] = dwagg[...] / (ridx + 1.0)
    t_o[:, _BASIS_DIM:] = jnp.where(col == 0, 1.0, 0.0).astype(_f32)


def _rk(rel, W0, W1, dwagg):
    return pl.pallas_call(
        _rk_body,
        out_shape=[
            jax.ShapeDtypeStruct((_NUM_REL, _ENT_DIM), _f32),
            jax.ShapeDtypeStruct((_NUM_REL, _ENT_DIM), _f32),
            jax.ShapeDtypeStruct((_NUM_REL, 128), _f32),
        ],
    )(rel, W0, W1, dwagg)


def kernel(edge_index, etype, norm, in_edges_mask, bp_edge_index, bp_etype,
           n_embds, rel_embds, degree_basis, degree_weights_agg,
           sub_W, sub_b, whp_W, whp_b,
           l0_WO, l0_WI, l0_WS, l0_WR, l0_loop, g0_W, g0_b,
           l1_WO, l1_WI, l1_WS, l1_WR, l1_loop, g1_W, g1_b):
    src3d = edge_index[0].reshape(_EROWS, 1, _ROW)
    dst3d = edge_index[1].reshape(_EROWS, 1, _ROW)
    et3d = etype.reshape(_EROWS, 1, _ROW)
    bpd3d = bp_edge_index[1].reshape(_EROWS, 1, _ROW)
    bpe3d = bp_etype.reshape(_EROWS, 1, _ROW)
    nmA = jnp.broadcast_to(norm.reshape(_N_EDGES, 1),
                           (_N_EDGES, 16)).reshape(_EROWS, _ROW, 16)
    nm_msk = norm.reshape(_N_EDGES, 1) * in_edges_mask.reshape(_N_EDGES, 1)
    nmM = jnp.broadcast_to(nm_msk, (_N_EDGES, 16)).reshape(_EROWS, _ROW, 16)
    z128 = jnp.zeros((632, 128), _f32)
    degones = jnp.broadcast_to(
        (jnp.arange(128) == _DEG_COL).astype(_f32), (_ROW, 128))

    r1, r2, ttbl = _rk(rel_embds, l0_WR, l1_WR, degree_weights_agg)

    bp2 = _sc1(bpd3d, bpe3d, ttbl, dst3d, z128, degones)

    swa, swb = sub_W[:_ENT_DIM], sub_W[_ENT_DIM:]
    sbr = sub_b.reshape(1, _ENT_DIM)
    nin, dns, dinvb = _tc1(bp2[0], bp2[1], n_embds, degree_basis,
                           swa, swb, sbr)

    rext0 = jnp.concatenate([rel_embds, l0_loop], axis=0)
    rext1 = jnp.concatenate([r1, l1_loop], axis=0)

    layers = [(l0_WO, l0_WI, l0_WS, l0_loop, g0_W, g0_b, rext0),
              (l1_WO, l1_WI, l1_WS, l1_loop, g1_W, g1_b, rext1)]
    nfin = None
    for li, (WO, WI, WS, lp, gW, gb, rext) in enumerate(layers):
        fcio, gout = _sc_edge(src3d, dst3d, et3d, nmA, nmM,
                              nin[:, :128], nin[:, 128:], dns,
                              rext[:, :128], rext[:, 128:], z128)
        accA = jnp.concatenate([fcio[0, 0], fcio[1, 0]], axis=1)
        accM = jnp.concatenate([fcio[0, 1], fcio[1, 1]], axis=1)
        accG = gout[0] + gout[1]
        gbr = gb.reshape(1, _INIT_DIM)
        if li == 0:
            nin, dns = _tc_layer(False, accA, accM, accG, nin, dns, dinvb,
                                 WI, WO, WS, lp, gW, gbr, swa, swb, sbr)
        else:
            nfin = _tc_layer(True, accA, accM, accG, nin, dns, dinvb,
                             WI, WO, WS, lp, gW, gbr,
                             whp_W[:_ENT_DIM], whp_W[_ENT_DIM:],
                             whp_b.reshape(1, _ENT_DIM))[0]
    return nfin, r2
